# baseline (speedup 1.0000x reference)
import functools

import jax
import jax.numpy as jnp
from jax import lax
from jax.experimental import pallas as pl
from jax.experimental.pallas import tpu as pltpu
from jax.experimental.pallas import tpu_sc as plsc

N = 10000
N_PAD = 10240
E = 320000
EROW = 64
ROWS = 5120
BLK = 512
GRID = N_PAD // BLK
NSUB = 16
NSL = N_PAD // NSUB

_mesh = plsc.VectorSubcoreMesh(core_axis_name="c", subcore_axis_name="s")


def _leaky(v):
    return jnp.where(v >= 0, v, 0.01 * v)


def _sc_degree(dst2d, ones16, zeros16):
    W = ones16.shape[1]

    @functools.partial(
        pl.kernel,
        out_type=jax.ShapeDtypeStruct((2, N_PAD, W), jnp.float32),
        mesh=_mesh,
        scratch_types=[
            pltpu.VMEM((CHUNK, EROW), jnp.int32),
            pltpu.VMEM((EROW, W), jnp.float32),
            pltpu.VMEM_SHARED((N_PAD, W), jnp.float32),
        ],
    )
    def k(dst_hbm, ones_hbm, z_hbm, out_hbm, didx, ones_v, acc_sh):
        cid = lax.axis_index("c")
        sid = lax.axis_index("s")
        nslice = pl.ds(sid * NSL, NSL)
        pltpu.sync_copy(z_hbm.at[nslice], acc_sh.at[nslice])
        pltpu.sync_copy(ones_hbm, ones_v)
        plsc.subcore_barrier()
        base = (cid * NSUB + sid) * (ROWS // 32)

        @pl.loop(0, ROWS // 32, step=CHUNK)
        def _(r):
            pltpu.sync_copy(dst_hbm.at[pl.ds(base + r, CHUNK)], didx)
            for j in range(CHUNK):
                pltpu.sync_copy(ones_v, acc_sh.at[didx.at[j]], add=True)

        plsc.subcore_barrier()
        pltpu.sync_copy(acc_sh.at[nslice], out_hbm.at[cid].at[nslice])

    return k(dst2d, ones16, zeros16)


NBUF = 4
CHUNK = 8


def _edge_loop(y_hbm, e_hbm, acc_sh, idx, rows, sems, base, nrows):
    @pl.loop(0, nrows, step=CHUNK)
    def _(r):
        pltpu.sync_copy(e_hbm.at[pl.ds(base + r, CHUNK)], idx)
        handles = {}
        for b in range(NBUF):
            handles[b] = pltpu.async_copy(y_hbm.at[idx.at[b, 0]], rows.at[b],
                                          sems[b])
        for j in range(CHUNK):
            handles[j].wait()
            pltpu.sync_copy(rows.at[j % NBUF], acc_sh.at[idx.at[j, 1]], add=True)
            nxt = j + NBUF
            if nxt < CHUNK:
                handles[nxt] = pltpu.async_copy(y_hbm.at[idx.at[nxt, 0]],
                                                rows.at[j % NBUF], sems[j % NBUF])


def _sc_scatter_split(y0, zeros, e2d):
    @functools.partial(
        pl.kernel,
        out_type=jax.ShapeDtypeStruct((2, N_PAD, 128), jnp.float32),
        mesh=_mesh,
        scratch_types=[
            pltpu.VMEM((CHUNK, 2, EROW), jnp.int32),
            pltpu.VMEM((NBUF, EROW, 128), jnp.float32),
            pltpu.VMEM_SHARED((N_PAD, 128), jnp.float32),
            pltpu.SemaphoreType.DMA,
            pltpu.SemaphoreType.DMA,
            pltpu.SemaphoreType.DMA,
            pltpu.SemaphoreType.DMA,
        ],
    )
    def k(y_hbm, z_hbm, e_hbm, out_hbm, idx, rows, acc_sh, s0, s1, s2, s3):
        sems = (s0, s1, s2, s3)
        cid = lax.axis_index("c")
        sid = lax.axis_index("s")
        nslice = pl.ds(sid * NSL, NSL)

        @pl.when(cid == 0)
        def _():
            pltpu.sync_copy(y_hbm.at[nslice], acc_sh.at[nslice])

        @pl.when(cid == 1)
        def _():
            pltpu.sync_copy(z_hbm.at[nslice], acc_sh.at[nslice])

        plsc.subcore_barrier()
        base = (cid * NSUB + sid) * (ROWS // 32)
        _edge_loop(y_hbm, e_hbm, acc_sh, idx, rows, sems, base, ROWS // 32)
        plsc.subcore_barrier()
        pltpu.sync_copy(acc_sh.at[nslice], out_hbm.at[cid].at[nslice])

    return k(y0, zeros, e2d)


def _sc_scatter_dual(y1, y2, e2d):
    @functools.partial(
        pl.kernel,
        out_type=jax.ShapeDtypeStruct((2, N_PAD, 128), jnp.float32),
        mesh=_mesh,
        scratch_types=[
            pltpu.VMEM((CHUNK, 2, EROW), jnp.int32),
            pltpu.VMEM((NBUF, EROW, 128), jnp.float32),
            pltpu.VMEM_SHARED((N_PAD, 128), jnp.float32),
            pltpu.SemaphoreType.DMA,
            pltpu.SemaphoreType.DMA,
            pltpu.SemaphoreType.DMA,
            pltpu.SemaphoreType.DMA,
        ],
    )
    def k(y1_hbm, y2_hbm, e_hbm, out_hbm, idx, rows, acc_sh, s0, s1, s2, s3):
        sems = (s0, s1, s2, s3)
        cid = lax.axis_index("c")
        sid = lax.axis_index("s")
        nslice = pl.ds(sid * NSL, NSL)
        base = sid * (ROWS // NSUB)

        @pl.when(cid == 0)
        def _():
            pltpu.sync_copy(y1_hbm.at[nslice], acc_sh.at[nslice])

        @pl.when(cid == 1)
        def _():
            pltpu.sync_copy(y2_hbm.at[nslice], acc_sh.at[nslice])

        plsc.subcore_barrier()

        @pl.when(cid == 0)
        def _():
            _edge_loop(y1_hbm, e_hbm, acc_sh, idx, rows, sems, base,
                       ROWS // NSUB)

        @pl.when(cid == 1)
        def _():
            _edge_loop(y2_hbm, e_hbm, acc_sh, idx, rows, sems, base,
                       ROWS // NSUB)

        plsc.subcore_barrier()
        pltpu.sync_copy(acc_sh.at[nslice], out_hbm.at[cid].at[nslice])

    return k(y1, y2, e2d)


def _dis_from(dg):
    deg = dg[0, :, 0:1] + dg[1, :, 0:1] + 1.0
    return lax.rsqrt(deg)


def _tc_y0(x_pad, w_shared, deg2):
    def body(x_ref, w_ref, dg_ref, y_ref):
        dis = _dis_from(dg_ref[...])
        xw = jnp.dot(x_ref[...], w_ref[...], preferred_element_type=jnp.float32)
        y_ref[...] = xw * dis

    return pl.pallas_call(
        body,
        grid=(GRID,),
        in_specs=[
            pl.BlockSpec((BLK, 128), lambda i: (i, 0)),
            pl.BlockSpec((128, 128), lambda i: (0, 0)),
            pl.BlockSpec((2, BLK, 128), lambda i: (0, i, 0)),
        ],
        out_specs=pl.BlockSpec((BLK, 128), lambda i: (i, 0)),
        out_shape=jax.ShapeDtypeStruct((N_PAD, 128), jnp.float32),
    )(x_pad, w_shared, deg2)


def _tc_mid(acc0, deg2, b_shared, w_c1, w_c2):
    def body(a_ref, dg_ref, b_ref, w1_ref, w2_ref, y1_ref, y2_ref):
        dis = _dis_from(dg_ref[...])
        a = a_ref[...]
        h = _leaky(dis * (a[0] + a[1]) + b_ref[...])
        y1_ref[...] = jnp.dot(h, w1_ref[...], preferred_element_type=jnp.float32) * dis
        y2_ref[...] = jnp.dot(h, w2_ref[...], preferred_element_type=jnp.float32) * dis

    return pl.pallas_call(
        body,
        grid=(GRID,),
        in_specs=[
            pl.BlockSpec((2, BLK, 128), lambda i: (0, i, 0)),
            pl.BlockSpec((2, BLK, 128), lambda i: (0, i, 0)),
            pl.BlockSpec((1, 128), lambda i: (0, 0)),
            pl.BlockSpec((128, 128), lambda i: (0, 0)),
            pl.BlockSpec((128, 128), lambda i: (0, 0)),
        ],
        out_specs=[
            pl.BlockSpec((BLK, 128), lambda i: (i, 0)),
            pl.BlockSpec((BLK, 128), lambda i: (i, 0)),
        ],
        out_shape=[
            jax.ShapeDtypeStruct((N_PAD, 128), jnp.float32),
            jax.ShapeDtypeStruct((N_PAD, 128), jnp.float32),
        ],
    )(acc0, deg2, b_shared, w_c1, w_c2)


def _tc_heads(accB, deg2, b_c1, b_c2, w_m1a, b_m1a, w_m1b, b_m1b,
              w_m2a, b_m2a, w_m2b, b_m2b):
    def body(a_ref, dg_ref, bc1, bc2, w1a, b1a, w1b, b1b, w2a, b2a, w2b, b2b,
             p1_ref, p2_ref):
        dis = _dis_from(dg_ref[...])
        a = a_ref[...]
        h1 = _leaky(dis * a[0] + bc1[...])
        t1 = _leaky(jnp.dot(h1, w1a[...], preferred_element_type=jnp.float32)
                    + b1a[...])
        p1_ref[...] = jnp.dot(t1, w1b[...], preferred_element_type=jnp.float32) + b1b[...]
        h2 = _leaky(dis * a[1] + bc2[...])
        t2 = _leaky(jnp.dot(h2, w2a[...], preferred_element_type=jnp.float32)
                    + b2a[...])
        p2_ref[...] = jnp.dot(t2, w2b[...], preferred_element_type=jnp.float32) + b2b[...]

    full = lambda r, c: pl.BlockSpec((r, c), lambda i: (0, 0))
    return pl.pallas_call(
        body,
        grid=(GRID,),
        in_specs=[
            pl.BlockSpec((2, BLK, 128), lambda i: (0, i, 0)),
            pl.BlockSpec((2, BLK, 128), lambda i: (0, i, 0)),
            full(1, 128), full(1, 128),
            full(128, 128), full(1, 128), full(128, 64), full(1, 64),
            full(128, 128), full(1, 128), full(128, 64), full(1, 64),
        ],
        out_specs=[
            pl.BlockSpec((BLK, 64), lambda i: (i, 0)),
            pl.BlockSpec((BLK, 64), lambda i: (i, 0)),
        ],
        out_shape=[
            jax.ShapeDtypeStruct((N_PAD, 64), jnp.float32),
            jax.ShapeDtypeStruct((N_PAD, 64), jnp.float32),
        ],
    )(accB, deg2, b_c1, b_c2, w_m1a, b_m1a, w_m1b, b_m1b,
      w_m2a, b_m2a, w_m2b, b_m2b)


def kernel(x, edge_index, w_shared, b_shared, w_c1, b_c1, w_c2, b_c2,
           w_m1a, b_m1a, w_m1b, b_m1b, w_m2a, b_m2a, w_m2b, b_m2b):
    f32 = jnp.float32
    src = edge_index[0].astype(jnp.int32)
    dst = edge_index[1].astype(jnp.int32)
    pad_e = ROWS * EROW - E
    fill = N + (jnp.arange(pad_e, dtype=jnp.int32) % (N_PAD - N))
    src2d = jnp.concatenate([src, fill]).reshape(ROWS, EROW)
    dst2d = jnp.concatenate([dst, fill]).reshape(ROWS, EROW)
    e2d = jnp.stack([src2d, dst2d], axis=1)
    x_pad = jnp.pad(x, ((0, N_PAD - N), (0, 0)))
    ones128 = jnp.ones((EROW, 128), f32)
    z128 = jnp.zeros((N_PAD, 128), f32)

    deg2 = _sc_degree(dst2d, ones128, z128)
    y0 = _tc_y0(x_pad, w_shared, deg2)
    acc0 = _sc_scatter_split(y0, z128, e2d)
    y1, y2 = _tc_mid(acc0, deg2, b_shared.reshape(1, -1), w_c1, w_c2)
    accB = _sc_scatter_dual(y1, y2, e2d)
    p1, p2 = _tc_heads(accB, deg2, b_c1.reshape(1, -1), b_c2.reshape(1, -1),
                       w_m1a, b_m1a.reshape(1, -1), w_m1b, b_m1b.reshape(1, -1),
                       w_m2a, b_m2a.reshape(1, -1), w_m2b, b_m2b.reshape(1, -1))
    return p1[:N], p2[:N]


# speedup vs baseline: 1.1023x; 1.1023x over previous
import functools

import jax
import jax.numpy as jnp
from jax import lax
from jax.experimental import pallas as pl
from jax.experimental.pallas import tpu as pltpu
from jax.experimental.pallas import tpu_sc as plsc

N = 10000
N_PAD = 10240
E = 320000
EROW = 128
ROWS = 2560
BLK = 512
GRID = N_PAD // BLK
NSUB = 16
NSL = N_PAD // NSUB

_mesh = plsc.VectorSubcoreMesh(core_axis_name="c", subcore_axis_name="s")


def _leaky(v):
    return jnp.where(v >= 0, v, 0.01 * v)


def _sc_degree(dst2d, ones16, zeros16):
    W = ones16.shape[1]

    @functools.partial(
        pl.kernel,
        out_type=jax.ShapeDtypeStruct((2, N_PAD, W), jnp.float32),
        mesh=_mesh,
        scratch_types=[
            pltpu.VMEM((CHUNK, EROW), jnp.int32),
            pltpu.VMEM((EROW, W), jnp.float32),
            pltpu.VMEM_SHARED((N_PAD, W), jnp.float32),
        ],
    )
    def k(dst_hbm, ones_hbm, z_hbm, out_hbm, didx, ones_v, acc_sh):
        cid = lax.axis_index("c")
        sid = lax.axis_index("s")
        nslice = pl.ds(sid * NSL, NSL)
        pltpu.sync_copy(z_hbm.at[nslice], acc_sh.at[nslice])
        pltpu.sync_copy(ones_hbm, ones_v)
        plsc.subcore_barrier()
        base = (cid * NSUB + sid) * (ROWS // 32)

        @pl.loop(0, ROWS // 32, step=CHUNK)
        def _(r):
            pltpu.sync_copy(dst_hbm.at[pl.ds(base + r, CHUNK)], didx)
            for j in range(CHUNK):
                pltpu.sync_copy(ones_v, acc_sh.at[didx.at[j]], add=True)

        plsc.subcore_barrier()
        pltpu.sync_copy(acc_sh.at[nslice], out_hbm.at[cid].at[nslice])

    return k(dst2d, ones16, zeros16)


NBUF = 2
CHUNK = 8


def _edge_loop(y_hbm, e_hbm, acc_sh, idx, rows, sems, base, nrows):
    @pl.loop(0, nrows, step=CHUNK)
    def _(r):
        pltpu.sync_copy(e_hbm.at[pl.ds(base + r, CHUNK)], idx)
        handles = {}
        for b in range(NBUF):
            handles[b] = pltpu.async_copy(y_hbm.at[idx.at[b, 0]], rows.at[b],
                                          sems[b])
        for j in range(CHUNK):
            handles[j].wait()
            pltpu.sync_copy(rows.at[j % NBUF], acc_sh.at[idx.at[j, 1]], add=True)
            nxt = j + NBUF
            if nxt < CHUNK:
                handles[nxt] = pltpu.async_copy(y_hbm.at[idx.at[nxt, 0]],
                                                rows.at[j % NBUF], sems[j % NBUF])


def _sc_scatter_split(y0, zeros, e2d):
    @functools.partial(
        pl.kernel,
        out_type=jax.ShapeDtypeStruct((2, N_PAD, 128), jnp.float32),
        mesh=_mesh,
        scratch_types=[
            pltpu.VMEM((CHUNK, 2, EROW), jnp.int32),
            pltpu.VMEM((NBUF, EROW, 128), jnp.float32),
            pltpu.VMEM_SHARED((N_PAD, 128), jnp.float32),
            pltpu.SemaphoreType.DMA,
            pltpu.SemaphoreType.DMA,
        ],
    )
    def k(y_hbm, z_hbm, e_hbm, out_hbm, idx, rows, acc_sh, s0, s1):
        sems = (s0, s1)
        cid = lax.axis_index("c")
        sid = lax.axis_index("s")
        nslice = pl.ds(sid * NSL, NSL)

        @pl.when(cid == 0)
        def _():
            pltpu.sync_copy(y_hbm.at[nslice], acc_sh.at[nslice])

        @pl.when(cid == 1)
        def _():
            pltpu.sync_copy(z_hbm.at[nslice], acc_sh.at[nslice])

        plsc.subcore_barrier()
        base = (cid * NSUB + sid) * (ROWS // 32)
        _edge_loop(y_hbm, e_hbm, acc_sh, idx, rows, sems, base, ROWS // 32)
        plsc.subcore_barrier()
        pltpu.sync_copy(acc_sh.at[nslice], out_hbm.at[cid].at[nslice])

    return k(y0, zeros, e2d)


def _sc_scatter_dual(y1, y2, e2d):
    @functools.partial(
        pl.kernel,
        out_type=jax.ShapeDtypeStruct((2, N_PAD, 128), jnp.float32),
        mesh=_mesh,
        scratch_types=[
            pltpu.VMEM((CHUNK, 2, EROW), jnp.int32),
            pltpu.VMEM((NBUF, EROW, 128), jnp.float32),
            pltpu.VMEM_SHARED((N_PAD, 128), jnp.float32),
            pltpu.SemaphoreType.DMA,
            pltpu.SemaphoreType.DMA,
        ],
    )
    def k(y1_hbm, y2_hbm, e_hbm, out_hbm, idx, rows, acc_sh, s0, s1):
        sems = (s0, s1)
        cid = lax.axis_index("c")
        sid = lax.axis_index("s")
        nslice = pl.ds(sid * NSL, NSL)
        base = sid * (ROWS // NSUB)

        @pl.when(cid == 0)
        def _():
            pltpu.sync_copy(y1_hbm.at[nslice], acc_sh.at[nslice])

        @pl.when(cid == 1)
        def _():
            pltpu.sync_copy(y2_hbm.at[nslice], acc_sh.at[nslice])

        plsc.subcore_barrier()

        @pl.when(cid == 0)
        def _():
            _edge_loop(y1_hbm, e_hbm, acc_sh, idx, rows, sems, base,
                       ROWS // NSUB)

        @pl.when(cid == 1)
        def _():
            _edge_loop(y2_hbm, e_hbm, acc_sh, idx, rows, sems, base,
                       ROWS // NSUB)

        plsc.subcore_barrier()
        pltpu.sync_copy(acc_sh.at[nslice], out_hbm.at[cid].at[nslice])

    return k(y1, y2, e2d)


def _dis_from(dg):
    deg = dg[0, :, 0:1] + dg[1, :, 0:1] + 1.0
    return lax.rsqrt(deg)


def _tc_xw(x_pad, w_shared):
    def body(x_ref, w_ref, o_ref):
        o_ref[...] = jnp.dot(x_ref[...], w_ref[...],
                             preferred_element_type=jnp.float32)

    return pl.pallas_call(
        body,
        grid=(GRID,),
        in_specs=[
            pl.BlockSpec((BLK, 128), lambda i: (i, 0)),
            pl.BlockSpec((128, 128), lambda i: (0, 0)),
        ],
        out_specs=pl.BlockSpec((BLK, 128), lambda i: (i, 0)),
        out_shape=jax.ShapeDtypeStruct((N_PAD, 128), jnp.float32),
    )(x_pad, w_shared)


def _tc_y0(xw, deg2):
    def body(xw_ref, dg_ref, y_ref):
        y_ref[...] = xw_ref[...] * _dis_from(dg_ref[...])

    return pl.pallas_call(
        body,
        grid=(GRID,),
        in_specs=[
            pl.BlockSpec((BLK, 128), lambda i: (i, 0)),
            pl.BlockSpec((2, BLK, 128), lambda i: (0, i, 0)),
        ],
        out_specs=pl.BlockSpec((BLK, 128), lambda i: (i, 0)),
        out_shape=jax.ShapeDtypeStruct((N_PAD, 128), jnp.float32),
    )(xw, deg2)


def _tc_mid(acc0, deg2, b_shared, w_c1, w_c2):
    def body(a_ref, dg_ref, b_ref, w1_ref, w2_ref, y1_ref, y2_ref):
        dis = _dis_from(dg_ref[...])
        a = a_ref[...]
        h = _leaky(dis * (a[0] + a[1]) + b_ref[...])
        y1_ref[...] = jnp.dot(h, w1_ref[...], preferred_element_type=jnp.float32) * dis
        y2_ref[...] = jnp.dot(h, w2_ref[...], preferred_element_type=jnp.float32) * dis

    return pl.pallas_call(
        body,
        grid=(GRID,),
        in_specs=[
            pl.BlockSpec((2, BLK, 128), lambda i: (0, i, 0)),
            pl.BlockSpec((2, BLK, 128), lambda i: (0, i, 0)),
            pl.BlockSpec((1, 128), lambda i: (0, 0)),
            pl.BlockSpec((128, 128), lambda i: (0, 0)),
            pl.BlockSpec((128, 128), lambda i: (0, 0)),
        ],
        out_specs=[
            pl.BlockSpec((BLK, 128), lambda i: (i, 0)),
            pl.BlockSpec((BLK, 128), lambda i: (i, 0)),
        ],
        out_shape=[
            jax.ShapeDtypeStruct((N_PAD, 128), jnp.float32),
            jax.ShapeDtypeStruct((N_PAD, 128), jnp.float32),
        ],
    )(acc0, deg2, b_shared, w_c1, w_c2)


def _tc_heads(accB, deg2, b_c1, b_c2, w_m1a, b_m1a, w_m1b, b_m1b,
              w_m2a, b_m2a, w_m2b, b_m2b):
    def body(a_ref, dg_ref, bc1, bc2, w1a, b1a, w1b, b1b, w2a, b2a, w2b, b2b,
             p1_ref, p2_ref):
        dis = _dis_from(dg_ref[...])
        a = a_ref[...]
        h1 = _leaky(dis * a[0] + bc1[...])
        t1 = _leaky(jnp.dot(h1, w1a[...], preferred_element_type=jnp.float32)
                    + b1a[...])
        p1_ref[...] = jnp.dot(t1, w1b[...], preferred_element_type=jnp.float32) + b1b[...]
        h2 = _leaky(dis * a[1] + bc2[...])
        t2 = _leaky(jnp.dot(h2, w2a[...], preferred_element_type=jnp.float32)
                    + b2a[...])
        p2_ref[...] = jnp.dot(t2, w2b[...], preferred_element_type=jnp.float32) + b2b[...]

    full = lambda r, c: pl.BlockSpec((r, c), lambda i: (0, 0))
    return pl.pallas_call(
        body,
        grid=(GRID,),
        in_specs=[
            pl.BlockSpec((2, BLK, 128), lambda i: (0, i, 0)),
            pl.BlockSpec((2, BLK, 128), lambda i: (0, i, 0)),
            full(1, 128), full(1, 128),
            full(128, 128), full(1, 128), full(128, 64), full(1, 64),
            full(128, 128), full(1, 128), full(128, 64), full(1, 64),
        ],
        out_specs=[
            pl.BlockSpec((BLK, 64), lambda i: (i, 0)),
            pl.BlockSpec((BLK, 64), lambda i: (i, 0)),
        ],
        out_shape=[
            jax.ShapeDtypeStruct((N, 64), jnp.float32),
            jax.ShapeDtypeStruct((N, 64), jnp.float32),
        ],
    )(accB, deg2, b_c1, b_c2, w_m1a, b_m1a, w_m1b, b_m1b,
      w_m2a, b_m2a, w_m2b, b_m2b)


def kernel(x, edge_index, w_shared, b_shared, w_c1, b_c1, w_c2, b_c2,
           w_m1a, b_m1a, w_m1b, b_m1b, w_m2a, b_m2a, w_m2b, b_m2b):
    f32 = jnp.float32
    src = edge_index[0].astype(jnp.int32)
    dst = edge_index[1].astype(jnp.int32)
    pad_e = ROWS * EROW - E
    fill = N + (jnp.arange(pad_e, dtype=jnp.int32) % (N_PAD - N))
    src2d = jnp.concatenate([src, fill]).reshape(ROWS, EROW)
    dst2d = jnp.concatenate([dst, fill]).reshape(ROWS, EROW)
    e2d = jnp.stack([src2d, dst2d], axis=1)
    x_pad = jnp.pad(x, ((0, N_PAD - N), (0, 0)))
    ones128 = jnp.ones((EROW, 128), f32)
    z128 = jnp.zeros((N_PAD, 128), f32)

    deg2 = _sc_degree(dst2d, ones128, z128)
    xw = _tc_xw(x_pad, w_shared)
    y0 = _tc_y0(xw, deg2)
    acc0 = _sc_scatter_split(y0, z128, e2d)
    y1, y2 = _tc_mid(acc0, deg2, b_shared.reshape(1, -1), w_c1, w_c2)
    accB = _sc_scatter_dual(y1, y2, e2d)
    p1, p2 = _tc_heads(accB, deg2, b_c1.reshape(1, -1), b_c2.reshape(1, -1),
                       w_m1a, b_m1a.reshape(1, -1), w_m1b, b_m1b.reshape(1, -1),
                       w_m2a, b_m2a.reshape(1, -1), w_m2b, b_m2b.reshape(1, -1))
    return p1, p2


# speedup vs baseline: 1.1111x; 1.0080x over previous
import functools

import jax
import jax.numpy as jnp
from jax import lax
from jax.experimental import pallas as pl
from jax.experimental.pallas import tpu as pltpu
from jax.experimental.pallas import tpu_sc as plsc

N = 10000
N_PAD = 10240
E = 320000
EROW = 128
ROWS = 2560
BLK = 512
GRID = N_PAD // BLK
NSUB = 16
NSL = N_PAD // NSUB

_mesh = plsc.VectorSubcoreMesh(core_axis_name="c", subcore_axis_name="s")


def _leaky(v):
    return jnp.where(v >= 0, v, 0.01 * v)


def _sc_degree(dst2d, ones16, zeros16):
    W = ones16.shape[1]

    @functools.partial(
        pl.kernel,
        out_type=jax.ShapeDtypeStruct((2, N_PAD, W), jnp.float32),
        mesh=_mesh,
        scratch_types=[
            pltpu.VMEM((CHUNK, EROW), jnp.int32),
            pltpu.VMEM((EROW, W), jnp.float32),
            pltpu.VMEM_SHARED((N_PAD, W), jnp.float32),
        ],
    )
    def k(dst_hbm, ones_hbm, z_hbm, out_hbm, didx, ones_v, acc_sh):
        cid = lax.axis_index("c")
        sid = lax.axis_index("s")
        nslice = pl.ds(sid * NSL, NSL)
        pltpu.sync_copy(z_hbm.at[nslice], acc_sh.at[nslice])
        pltpu.sync_copy(ones_hbm, ones_v)
        plsc.subcore_barrier()
        base = (cid * NSUB + sid) * (ROWS // 32)

        @pl.loop(0, ROWS // 32, step=CHUNK)
        def _(r):
            pltpu.sync_copy(dst_hbm.at[pl.ds(base + r, CHUNK)], didx)
            for j in range(CHUNK):
                pltpu.sync_copy(ones_v, acc_sh.at[didx.at[j]], add=True)

        plsc.subcore_barrier()
        pltpu.sync_copy(acc_sh.at[nslice], out_hbm.at[cid].at[nslice])

    return k(dst2d, ones16, zeros16)


NBUF = 2
CHUNK = 8


def _edge_loop(y_hbm, e_hbm, acc_sh, idx, rows, sems, base, nrows):
    @pl.loop(0, nrows, step=CHUNK)
    def _(r):
        pltpu.sync_copy(e_hbm.at[pl.ds(base + r, CHUNK)], idx)
        handles = {}
        for b in range(NBUF):
            handles[b] = pltpu.async_copy(y_hbm.at[idx.at[b, 0]], rows.at[b],
                                          sems[b])
        for j in range(CHUNK):
            handles[j].wait()
            pltpu.sync_copy(rows.at[j % NBUF], acc_sh.at[idx.at[j, 1]], add=True)
            nxt = j + NBUF
            if nxt < CHUNK:
                handles[nxt] = pltpu.async_copy(y_hbm.at[idx.at[nxt, 0]],
                                                rows.at[j % NBUF], sems[j % NBUF])


def _sc_scatter_split(y0, zeros, e2d):
    @functools.partial(
        pl.kernel,
        out_type=jax.ShapeDtypeStruct((2, N_PAD, 128), jnp.float32),
        mesh=_mesh,
        scratch_types=[
            pltpu.VMEM((CHUNK, 2, EROW), jnp.int32),
            pltpu.VMEM((NBUF, EROW, 128), jnp.float32),
            pltpu.VMEM_SHARED((N_PAD, 128), jnp.float32),
            pltpu.SemaphoreType.DMA,
            pltpu.SemaphoreType.DMA,
        ],
    )
    def k(y_hbm, z_hbm, e_hbm, out_hbm, idx, rows, acc_sh, s0, s1):
        sems = (s0, s1)
        cid = lax.axis_index("c")
        sid = lax.axis_index("s")
        nslice = pl.ds(sid * NSL, NSL)

        @pl.when(cid == 0)
        def _():
            pltpu.sync_copy(y_hbm.at[nslice], acc_sh.at[nslice])

        @pl.when(cid == 1)
        def _():
            pltpu.sync_copy(z_hbm.at[nslice], acc_sh.at[nslice])

        plsc.subcore_barrier()
        base = (cid * NSUB + sid) * (ROWS // 32)
        _edge_loop(y_hbm, e_hbm, acc_sh, idx, rows, sems, base, ROWS // 32)
        plsc.subcore_barrier()
        pltpu.sync_copy(acc_sh.at[nslice], out_hbm.at[cid].at[nslice])

    return k(y0, zeros, e2d)


def _sc_scatter_dual(y1, y2, e2d):
    @functools.partial(
        pl.kernel,
        out_type=jax.ShapeDtypeStruct((2, N_PAD, 128), jnp.float32),
        mesh=_mesh,
        scratch_types=[
            pltpu.VMEM((CHUNK, 2, EROW), jnp.int32),
            pltpu.VMEM((NBUF, EROW, 128), jnp.float32),
            pltpu.VMEM_SHARED((N_PAD, 128), jnp.float32),
            pltpu.SemaphoreType.DMA,
            pltpu.SemaphoreType.DMA,
        ],
    )
    def k(y1_hbm, y2_hbm, e_hbm, out_hbm, idx, rows, acc_sh, s0, s1):
        sems = (s0, s1)
        cid = lax.axis_index("c")
        sid = lax.axis_index("s")
        nslice = pl.ds(sid * NSL, NSL)
        base = sid * (ROWS // NSUB)

        @pl.when(cid == 0)
        def _():
            pltpu.sync_copy(y1_hbm.at[nslice], acc_sh.at[nslice])

        @pl.when(cid == 1)
        def _():
            pltpu.sync_copy(y2_hbm.at[nslice], acc_sh.at[nslice])

        plsc.subcore_barrier()

        @pl.when(cid == 0)
        def _():
            _edge_loop(y1_hbm, e_hbm, acc_sh, idx, rows, sems, base,
                       ROWS // NSUB)

        @pl.when(cid == 1)
        def _():
            _edge_loop(y2_hbm, e_hbm, acc_sh, idx, rows, sems, base,
                       ROWS // NSUB)

        plsc.subcore_barrier()
        pltpu.sync_copy(acc_sh.at[nslice], out_hbm.at[cid].at[nslice])

    return k(y1, y2, e2d)


def _dis_from(dg):
    deg = dg[0, :, 0:1] + dg[1, :, 0:1] + 1.0
    return lax.rsqrt(deg)


def _tc_xw(x_pad, w_shared):
    def body(x_ref, w_ref, o_ref):
        o_ref[...] = jnp.dot(x_ref[...], w_ref[...],
                             preferred_element_type=jnp.float32)

    return pl.pallas_call(
        body,
        grid=(GRID,),
        in_specs=[
            pl.BlockSpec((BLK, 128), lambda i: (i, 0)),
            pl.BlockSpec((128, 128), lambda i: (0, 0)),
        ],
        out_specs=pl.BlockSpec((BLK, 128), lambda i: (i, 0)),
        out_shape=jax.ShapeDtypeStruct((N_PAD, 128), jnp.float32),
    )(x_pad, w_shared)


def _tc_y0(xw, deg2):
    def body(xw_ref, dg_ref, y_ref, d_ref):
        dis = _dis_from(dg_ref[...])
        y_ref[...] = xw_ref[...] * dis
        d_ref[...] = jnp.broadcast_to(dis, (BLK, 16))

    return pl.pallas_call(
        body,
        grid=(GRID,),
        in_specs=[
            pl.BlockSpec((BLK, 128), lambda i: (i, 0)),
            pl.BlockSpec((2, BLK, 128), lambda i: (0, i, 0)),
        ],
        out_specs=[
            pl.BlockSpec((BLK, 128), lambda i: (i, 0)),
            pl.BlockSpec((BLK, 16), lambda i: (i, 0)),
        ],
        out_shape=[
            jax.ShapeDtypeStruct((N_PAD, 128), jnp.float32),
            jax.ShapeDtypeStruct((N_PAD, 16), jnp.float32),
        ],
    )(xw, deg2)


def _tc_mid(acc0, dis16, b_shared, w_c1, w_c2):
    def body(a_ref, dg_ref, b_ref, w1_ref, w2_ref, y1_ref, y2_ref):
        dis = dg_ref[:, 0:1]
        a = a_ref[...]
        h = _leaky(dis * (a[0] + a[1]) + b_ref[...])
        y1_ref[...] = jnp.dot(h, w1_ref[...], preferred_element_type=jnp.float32) * dis
        y2_ref[...] = jnp.dot(h, w2_ref[...], preferred_element_type=jnp.float32) * dis

    return pl.pallas_call(
        body,
        grid=(GRID,),
        in_specs=[
            pl.BlockSpec((2, BLK, 128), lambda i: (0, i, 0)),
            pl.BlockSpec((BLK, 16), lambda i: (i, 0)),
            pl.BlockSpec((1, 128), lambda i: (0, 0)),
            pl.BlockSpec((128, 128), lambda i: (0, 0)),
            pl.BlockSpec((128, 128), lambda i: (0, 0)),
        ],
        out_specs=[
            pl.BlockSpec((BLK, 128), lambda i: (i, 0)),
            pl.BlockSpec((BLK, 128), lambda i: (i, 0)),
        ],
        out_shape=[
            jax.ShapeDtypeStruct((N_PAD, 128), jnp.float32),
            jax.ShapeDtypeStruct((N_PAD, 128), jnp.float32),
        ],
    )(acc0, dis16, b_shared, w_c1, w_c2)


def _tc_heads(accB, dis16, b_c1, b_c2, w_m1a, b_m1a, w_m1b, b_m1b,
              w_m2a, b_m2a, w_m2b, b_m2b):
    def body(a_ref, dg_ref, bc1, bc2, w1a, b1a, w1b, b1b, w2a, b2a, w2b, b2b,
             p1_ref, p2_ref):
        dis = dg_ref[:, 0:1]
        a = a_ref[...]
        h1 = _leaky(dis * a[0] + bc1[...])
        t1 = _leaky(jnp.dot(h1, w1a[...], preferred_element_type=jnp.float32)
                    + b1a[...])
        p1_ref[...] = jnp.dot(t1, w1b[...], preferred_element_type=jnp.float32) + b1b[...]
        h2 = _leaky(dis * a[1] + bc2[...])
        t2 = _leaky(jnp.dot(h2, w2a[...], preferred_element_type=jnp.float32)
                    + b2a[...])
        p2_ref[...] = jnp.dot(t2, w2b[...], preferred_element_type=jnp.float32) + b2b[...]

    full = lambda r, c: pl.BlockSpec((r, c), lambda i: (0, 0))
    return pl.pallas_call(
        body,
        grid=(GRID,),
        in_specs=[
            pl.BlockSpec((2, BLK, 128), lambda i: (0, i, 0)),
            pl.BlockSpec((BLK, 16), lambda i: (i, 0)),
            full(1, 128), full(1, 128),
            full(128, 128), full(1, 128), full(128, 64), full(1, 64),
            full(128, 128), full(1, 128), full(128, 64), full(1, 64),
        ],
        out_specs=[
            pl.BlockSpec((BLK, 64), lambda i: (i, 0)),
            pl.BlockSpec((BLK, 64), lambda i: (i, 0)),
        ],
        out_shape=[
            jax.ShapeDtypeStruct((N, 64), jnp.float32),
            jax.ShapeDtypeStruct((N, 64), jnp.float32),
        ],
    )(accB, dis16, b_c1, b_c2, w_m1a, b_m1a, w_m1b, b_m1b,
      w_m2a, b_m2a, w_m2b, b_m2b)


def kernel(x, edge_index, w_shared, b_shared, w_c1, b_c1, w_c2, b_c2,
           w_m1a, b_m1a, w_m1b, b_m1b, w_m2a, b_m2a, w_m2b, b_m2b):
    f32 = jnp.float32
    src = edge_index[0].astype(jnp.int32)
    dst = edge_index[1].astype(jnp.int32)
    pad_e = ROWS * EROW - E
    fill = N + (jnp.arange(pad_e, dtype=jnp.int32) % (N_PAD - N))
    src2d = jnp.concatenate([src, fill]).reshape(ROWS, EROW)
    dst2d = jnp.concatenate([dst, fill]).reshape(ROWS, EROW)
    e2d = jnp.stack([src2d, dst2d], axis=1)
    x_pad = jnp.pad(x, ((0, N_PAD - N), (0, 0)))
    ones128 = jnp.ones((EROW, 128), f32)
    z128 = jnp.zeros((N_PAD, 128), f32)

    deg2 = _sc_degree(dst2d, ones128, z128)
    xw = _tc_xw(x_pad, w_shared)
    y0, dis16 = _tc_y0(xw, deg2)
    acc0 = _sc_scatter_split(y0, z128, e2d)
    y1, y2 = _tc_mid(acc0, dis16, b_shared.reshape(1, -1), w_c1, w_c2)
    accB = _sc_scatter_dual(y1, y2, e2d)
    p1, p2 = _tc_heads(accB, dis16, b_c1.reshape(1, -1), b_c2.reshape(1, -1),
                       w_m1a, b_m1a.reshape(1, -1), w_m1b, b_m1b.reshape(1, -1),
                       w_m2a, b_m2a.reshape(1, -1), w_m2b, b_m2b.reshape(1, -1))
    return p1, p2


# speedup vs baseline: 1.2374x; 1.1137x over previous
import dataclasses
import functools

import jax
import jax.numpy as jnp
from jax import lax
from jax.experimental import pallas as pl
from jax.experimental.pallas import tpu as pltpu
from jax.experimental.pallas import tpu_sc as plsc

N = 10000
N_PAD = 10240
E = 320000
EROW = 128
ROWS = 2560
BLK = 512
GRID = N_PAD // BLK
NSUB = 16
NSL = N_PAD // NSUB

_mesh = plsc.VectorSubcoreMesh(core_axis_name="c", subcore_axis_name="s")


def _leaky(v):
    return jnp.where(v >= 0, v, 0.01 * v)


def _sc_degree(dst2d, zeros1):
    cp = pltpu.CompilerParams()
    if "needs_layout_passes" in pltpu.CompilerParams.__dataclass_fields__:
        cp = dataclasses.replace(cp, needs_layout_passes=False)

    @functools.partial(
        pl.kernel,
        out_type=jax.ShapeDtypeStruct((32, N_PAD), jnp.float32),
        mesh=_mesh,
        scratch_types=[
            pltpu.VMEM((CHUNK, EROW), jnp.int32),
            pltpu.VMEM((N_PAD,), jnp.float32),
        ],
        compiler_params=cp,
    )
    def k(dst_hbm, z_hbm, out_hbm, didx, cnt):
        cid = lax.axis_index("c")
        sid = lax.axis_index("s")
        wid = cid * NSUB + sid
        pltpu.sync_copy(z_hbm, cnt)
        ones16 = jnp.ones((16,), jnp.float32)
        base = wid * (ROWS // 32)

        @pl.loop(0, ROWS // 32, step=CHUNK)
        def _(r):
            pltpu.sync_copy(dst_hbm.at[pl.ds(base + r, CHUNK)], didx)
            for j in range(CHUNK):
                for g in range(EROW // 16):
                    idx16 = didx[j, pl.ds(g * 16, 16)]
                    plsc.addupdate_scatter(cnt, [idx16], ones16)

        pltpu.sync_copy(cnt, out_hbm.at[wid])

    return k(dst2d, zeros1)


NBUF = 2
CHUNK = 8


def _edge_loop(y_hbm, e_hbm, acc_sh, idx, rows, sems, base, nrows):
    @pl.loop(0, nrows, step=CHUNK)
    def _(r):
        pltpu.sync_copy(e_hbm.at[pl.ds(base + r, CHUNK)], idx)
        handles = {}
        for b in range(NBUF):
            handles[b] = pltpu.async_copy(y_hbm.at[idx.at[b, 0]], rows.at[b],
                                          sems[b])
        for j in range(CHUNK):
            handles[j].wait()
            pltpu.sync_copy(rows.at[j % NBUF], acc_sh.at[idx.at[j, 1]], add=True)
            nxt = j + NBUF
            if nxt < CHUNK:
                handles[nxt] = pltpu.async_copy(y_hbm.at[idx.at[nxt, 0]],
                                                rows.at[j % NBUF], sems[j % NBUF])


def _sc_scatter_split(y0, zeros, e2d):
    @functools.partial(
        pl.kernel,
        out_type=jax.ShapeDtypeStruct((2, N_PAD, 128), jnp.float32),
        mesh=_mesh,
        scratch_types=[
            pltpu.VMEM((CHUNK, 2, EROW), jnp.int32),
            pltpu.VMEM((NBUF, EROW, 128), jnp.float32),
            pltpu.VMEM_SHARED((N_PAD, 128), jnp.float32),
            pltpu.SemaphoreType.DMA,
            pltpu.SemaphoreType.DMA,
        ],
    )
    def k(y_hbm, z_hbm, e_hbm, out_hbm, idx, rows, acc_sh, s0, s1):
        sems = (s0, s1)
        cid = lax.axis_index("c")
        sid = lax.axis_index("s")
        nslice = pl.ds(sid * NSL, NSL)

        @pl.when(cid == 0)
        def _():
            pltpu.sync_copy(y_hbm.at[nslice], acc_sh.at[nslice])

        @pl.when(cid == 1)
        def _():
            pltpu.sync_copy(z_hbm.at[nslice], acc_sh.at[nslice])

        plsc.subcore_barrier()
        base = (cid * NSUB + sid) * (ROWS // 32)
        _edge_loop(y_hbm, e_hbm, acc_sh, idx, rows, sems, base, ROWS // 32)
        plsc.subcore_barrier()
        pltpu.sync_copy(acc_sh.at[nslice], out_hbm.at[cid].at[nslice])

    return k(y0, zeros, e2d)


def _sc_scatter_dual(y1, y2, e2d):
    @functools.partial(
        pl.kernel,
        out_type=jax.ShapeDtypeStruct((2, N_PAD, 128), jnp.float32),
        mesh=_mesh,
        scratch_types=[
            pltpu.VMEM((CHUNK, 2, EROW), jnp.int32),
            pltpu.VMEM((NBUF, EROW, 128), jnp.float32),
            pltpu.VMEM_SHARED((N_PAD, 128), jnp.float32),
            pltpu.SemaphoreType.DMA,
            pltpu.SemaphoreType.DMA,
        ],
    )
    def k(y1_hbm, y2_hbm, e_hbm, out_hbm, idx, rows, acc_sh, s0, s1):
        sems = (s0, s1)
        cid = lax.axis_index("c")
        sid = lax.axis_index("s")
        nslice = pl.ds(sid * NSL, NSL)
        base = sid * (ROWS // NSUB)

        @pl.when(cid == 0)
        def _():
            pltpu.sync_copy(y1_hbm.at[nslice], acc_sh.at[nslice])

        @pl.when(cid == 1)
        def _():
            pltpu.sync_copy(y2_hbm.at[nslice], acc_sh.at[nslice])

        plsc.subcore_barrier()

        @pl.when(cid == 0)
        def _():
            _edge_loop(y1_hbm, e_hbm, acc_sh, idx, rows, sems, base,
                       ROWS // NSUB)

        @pl.when(cid == 1)
        def _():
            _edge_loop(y2_hbm, e_hbm, acc_sh, idx, rows, sems, base,
                       ROWS // NSUB)

        plsc.subcore_barrier()
        pltpu.sync_copy(acc_sh.at[nslice], out_hbm.at[cid].at[nslice])

    return k(y1, y2, e2d)


def _tc_xw(x_pad, w_shared):
    def body(x_ref, w_ref, o_ref):
        o_ref[...] = jnp.dot(x_ref[...], w_ref[...],
                             preferred_element_type=jnp.float32)

    return pl.pallas_call(
        body,
        grid=(GRID,),
        in_specs=[
            pl.BlockSpec((BLK, 128), lambda i: (i, 0)),
            pl.BlockSpec((128, 128), lambda i: (0, 0)),
        ],
        out_specs=pl.BlockSpec((BLK, 128), lambda i: (i, 0)),
        out_shape=jax.ShapeDtypeStruct((N_PAD, 128), jnp.float32),
    )(x_pad, w_shared)


def _tc_y0(xw, deg2):
    def body(xw_ref, dg_ref, y_ref, d_ref):
        deg = jnp.sum(dg_ref[...], axis=0).reshape(BLK, 1) + 1.0
        dis = lax.rsqrt(deg)
        y_ref[...] = xw_ref[...] * dis
        d_ref[...] = jnp.broadcast_to(dis, (BLK, 16))

    return pl.pallas_call(
        body,
        grid=(GRID,),
        in_specs=[
            pl.BlockSpec((BLK, 128), lambda i: (i, 0)),
            pl.BlockSpec((32, BLK), lambda i: (0, i)),
        ],
        out_specs=[
            pl.BlockSpec((BLK, 128), lambda i: (i, 0)),
            pl.BlockSpec((BLK, 16), lambda i: (i, 0)),
        ],
        out_shape=[
            jax.ShapeDtypeStruct((N_PAD, 128), jnp.float32),
            jax.ShapeDtypeStruct((N_PAD, 16), jnp.float32),
        ],
    )(xw, deg2)


def _tc_mid(acc0, dis16, b_shared, w_c1, w_c2):
    def body(a_ref, dg_ref, b_ref, w1_ref, w2_ref, y1_ref, y2_ref):
        dis = dg_ref[:, 0:1]
        a = a_ref[...]
        h = _leaky(dis * (a[0] + a[1]) + b_ref[...])
        y1_ref[...] = jnp.dot(h, w1_ref[...], preferred_element_type=jnp.float32) * dis
        y2_ref[...] = jnp.dot(h, w2_ref[...], preferred_element_type=jnp.float32) * dis

    return pl.pallas_call(
        body,
        grid=(GRID,),
        in_specs=[
            pl.BlockSpec((2, BLK, 128), lambda i: (0, i, 0)),
            pl.BlockSpec((BLK, 16), lambda i: (i, 0)),
            pl.BlockSpec((1, 128), lambda i: (0, 0)),
            pl.BlockSpec((128, 128), lambda i: (0, 0)),
            pl.BlockSpec((128, 128), lambda i: (0, 0)),
        ],
        out_specs=[
            pl.BlockSpec((BLK, 128), lambda i: (i, 0)),
            pl.BlockSpec((BLK, 128), lambda i: (i, 0)),
        ],
        out_shape=[
            jax.ShapeDtypeStruct((N_PAD, 128), jnp.float32),
            jax.ShapeDtypeStruct((N_PAD, 128), jnp.float32),
        ],
    )(acc0, dis16, b_shared, w_c1, w_c2)


def _tc_heads(accB, dis16, b_c1, b_c2, w_m1a, b_m1a, w_m1b, b_m1b,
              w_m2a, b_m2a, w_m2b, b_m2b):
    def body(a_ref, dg_ref, bc1, bc2, w1a, b1a, w1b, b1b, w2a, b2a, w2b, b2b,
             p1_ref, p2_ref):
        dis = dg_ref[:, 0:1]
        a = a_ref[...]
        h1 = _leaky(dis * a[0] + bc1[...])
        t1 = _leaky(jnp.dot(h1, w1a[...], preferred_element_type=jnp.float32)
                    + b1a[...])
        p1_ref[...] = jnp.dot(t1, w1b[...], preferred_element_type=jnp.float32) + b1b[...]
        h2 = _leaky(dis * a[1] + bc2[...])
        t2 = _leaky(jnp.dot(h2, w2a[...], preferred_element_type=jnp.float32)
                    + b2a[...])
        p2_ref[...] = jnp.dot(t2, w2b[...], preferred_element_type=jnp.float32) + b2b[...]

    full = lambda r, c: pl.BlockSpec((r, c), lambda i: (0, 0))
    return pl.pallas_call(
        body,
        grid=(GRID,),
        in_specs=[
            pl.BlockSpec((2, BLK, 128), lambda i: (0, i, 0)),
            pl.BlockSpec((BLK, 16), lambda i: (i, 0)),
            full(1, 128), full(1, 128),
            full(128, 128), full(1, 128), full(128, 64), full(1, 64),
            full(128, 128), full(1, 128), full(128, 64), full(1, 64),
        ],
        out_specs=[
            pl.BlockSpec((BLK, 64), lambda i: (i, 0)),
            pl.BlockSpec((BLK, 64), lambda i: (i, 0)),
        ],
        out_shape=[
            jax.ShapeDtypeStruct((N, 64), jnp.float32),
            jax.ShapeDtypeStruct((N, 64), jnp.float32),
        ],
    )(accB, dis16, b_c1, b_c2, w_m1a, b_m1a, w_m1b, b_m1b,
      w_m2a, b_m2a, w_m2b, b_m2b)


def kernel(x, edge_index, w_shared, b_shared, w_c1, b_c1, w_c2, b_c2,
           w_m1a, b_m1a, w_m1b, b_m1b, w_m2a, b_m2a, w_m2b, b_m2b):
    f32 = jnp.float32
    src = edge_index[0].astype(jnp.int32)
    dst = edge_index[1].astype(jnp.int32)
    pad_e = ROWS * EROW - E
    fill = N + (jnp.arange(pad_e, dtype=jnp.int32) % (N_PAD - N))
    src2d = jnp.concatenate([src, fill]).reshape(ROWS, EROW)
    dst2d = jnp.concatenate([dst, fill]).reshape(ROWS, EROW)
    e2d = jnp.stack([src2d, dst2d], axis=1)
    x_pad = jnp.pad(x, ((0, N_PAD - N), (0, 0)))
    z128 = jnp.zeros((N_PAD, 128), f32)

    deg2 = _sc_degree(dst2d, jnp.zeros((N_PAD,), f32))
    xw = _tc_xw(x_pad, w_shared)
    y0, dis16 = _tc_y0(xw, deg2)
    acc0 = _sc_scatter_split(y0, z128, e2d)
    y1, y2 = _tc_mid(acc0, dis16, b_shared.reshape(1, -1), w_c1, w_c2)
    accB = _sc_scatter_dual(y1, y2, e2d)
    p1, p2 = _tc_heads(accB, dis16, b_c1.reshape(1, -1), b_c2.reshape(1, -1),
                       w_m1a, b_m1a.reshape(1, -1), w_m1b, b_m1b.reshape(1, -1),
                       w_m2a, b_m2a.reshape(1, -1), w_m2b, b_m2b.reshape(1, -1))
    return p1, p2
